# baseline (speedup 1.0000x reference)
import functools

import jax
import jax.numpy as jnp
from jax import lax
from jax.experimental import pallas as pl
from jax.experimental.pallas import tpu as pltpu
from jax.experimental.pallas import tpu_sc as plsc

IN_DIM = 4096
OUT_DIM = 8192
BATCH = 4096

NC = 2
NS = 16
NW = NC * NS
L = 16

ROWS_PER_TILE = BATCH // NW
R = 8
NBLK = ROWS_PER_TILE // R
S = 1024
NSTRIP = OUT_DIM // S
CPS = S // L
CW = 128
NCW = OUT_DIM // CW


def _round_bf16_bits(v):
    bits = plsc.bitcast(v, jnp.uint32)
    return lax.shift_right_logical(bits + jnp.uint32(0x8000),
                                   jnp.uint32(16))


def _coef_body(w_cols):
    m = w_cols[0]
    for g in range(1, 16):
        m = jnp.maximum(m, w_cols[g])
    e = [jnp.exp(w_cols[g] - m) for g in range(16)]
    s = e[0]
    for g in range(1, 16):
        s = s + e[g]
    inv = 1.0 / s
    ca = (e[2] + e[3]) + (e[6] + e[7]) - (e[8] + e[9]) - (e[12] + e[13])
    cb = (e[4] + e[5]) + (e[6] + e[7]) - (e[8] + e[9]) - (e[10] + e[11])
    cab = (e[1] - e[2]) - (e[4] + e[7]) - 2.0 * (e[6] - e[9]) \
        + (e[8] + e[11]) + (e[13] - e[14])
    c0 = (e[8] + e[9]) + (e[10] + e[11]) + (e[12] + e[13]) + (e[14] + e[15])
    return c0 * inv, ca * inv, cb * inv, cab * inv


def _unpack_pair(pk):
    lo = plsc.bitcast(lax.shift_left(pk, jnp.uint32(16)), jnp.float32)
    hi = plsc.bitcast(pk & jnp.uint32(0xFFFF0000), jnp.float32)
    return lo, hi


SLC = OUT_DIM // NS


def _tec_body(x_hbm, wt_hbm, ia_hbm, ib_hbm, out_hbm,
              iab_v, stage_v, cp0_v, cp1_v, wbuf_v, xb0_v, xb1_v, obuf_v,
              iab_s, cp0_s, cp1_s, sem_x, sem_o):
    cid = lax.axis_index("c")
    sid = lax.axis_index("s")
    wid = sid * NC + cid
    row0 = wid * ROWS_PER_TILE

    pltpu.async_copy(
        x_hbm.at[pl.ds(row0 * IN_DIM, R * IN_DIM)], xb0_v, sem_x)

    off = sid * SLC

    pltpu.sync_copy(ia_hbm.at[pl.ds(off, SLC)], iab_v.at[pl.ds(off, SLC)])
    pltpu.sync_copy(ib_hbm.at[pl.ds(off, SLC)], stage_v)

    def pack_idx(j, carry):
        ds = pl.ds(off + j * L, L)
        iab_v[ds] = iab_v[ds] | \
            lax.shift_left(stage_v[pl.ds(j * L, L)], 16)
        return carry

    lax.fori_loop(0, SLC // L, pack_idx, 0)
    pltpu.sync_copy(iab_v.at[pl.ds(off, SLC)], iab_s.at[pl.ds(off, SLC)])

    def coef_chunk(k, carry):
        c = sid * (SLC // CW) + k
        pltpu.sync_copy(wt_hbm.at[:, pl.ds(c * CW, CW)], wbuf_v)

        def coef_group(j, carry2):
            w_cols = [wbuf_v[g, pl.ds(j * L, L)] for g in range(16)]
            c0, ca, cb, cab = _coef_body(w_cols)
            base = pl.ds(c * CW + j * L, L)
            cp0_v[base] = _round_bf16_bits(c0) | \
                lax.shift_left(_round_bf16_bits(ca), jnp.uint32(16))
            cp1_v[base] = _round_bf16_bits(cb) | \
                lax.shift_left(_round_bf16_bits(cab), jnp.uint32(16))
            return carry2

        lax.fori_loop(0, CW // L, coef_group, 0)
        return carry

    lax.fori_loop(0, SLC // CW, coef_chunk, 0)
    pltpu.sync_copy(cp0_v.at[pl.ds(off, SLC)], cp0_s.at[pl.ds(off, SLC)])
    pltpu.sync_copy(cp1_v.at[pl.ds(off, SLC)], cp1_s.at[pl.ds(off, SLC)])

    plsc.subcore_barrier()

    pltpu.sync_copy(iab_s, iab_v)
    pltpu.sync_copy(cp0_s, cp0_v)
    pltpu.sync_copy(cp1_s, cp1_v)


    xbufs = (xb0_v, xb1_v)

    def row_pair(i, carry):
        for half in range(2):
            blk = 2 * i + half
            rbase = row0 + blk * R
            xb = xbufs[half]
            pltpu.make_async_copy(
                x_hbm.at[pl.ds(rbase * IN_DIM, R * IN_DIM)], xb,
                sem_x).wait()

            @pl.when(blk + 1 < NBLK)
            def _(rbase=rbase, half=half):
                pltpu.async_copy(
                    x_hbm.at[pl.ds((rbase + R) * IN_DIM, R * IN_DIM)],
                    xbufs[1 - half], sem_x)

            def strip(st, carry2, blk=blk, rbase=rbase, xb=xb):
                pb = st & 1
                gst = blk * NSTRIP + st
                obase = st * S

                @pl.when(gst >= 2)
                def _():
                    pltpu.make_async_copy(
                        obuf_v.at[pb],
                        out_hbm.at[pl.ds(rbase, R), pl.ds(obase, S)],
                        sem_o).wait()

                @plsc.parallel_loop(0, CPS, 1, unroll=2)
                def chunk(oc, xb=xb):
                    ds = pl.ds(obase + oc * L, L)
                    pk = iab_v[ds]
                    ia = pk & jnp.int32(0xFFFF)
                    ib = lax.shift_right_logical(pk, 16)
                    c0, ca = _unpack_pair(cp0_v[ds])
                    cb, cab = _unpack_pair(cp1_v[ds])
                    for r in range(R):
                        row = pl.ds(r * IN_DIM, IN_DIM)
                        a = plsc.load_gather(xb.at[row], [ia])
                        b = plsc.load_gather(xb.at[row], [ib])
                        obuf_v[pb, r, pl.ds(oc * L, L)] = \
                            (c0 + ca * a) + (cb + cab * a) * b

                pltpu.async_copy(
                    obuf_v.at[pb],
                    out_hbm.at[pl.ds(rbase, R), pl.ds(obase, S)],
                    sem_o)
                return carry2

            lax.fori_loop(0, NSTRIP, strip, 0)
        return carry

    lax.fori_loop(0, NBLK // 2, row_pair, 0)

    for _ in range(2):
        pltpu.make_async_copy(
            obuf_v.at[0],
            out_hbm.at[pl.ds(row0, R), pl.ds(0, S)],
            sem_o).wait()


@jax.jit
def _logic_layer_sc(x, wt, idx_a, idx_b):
    mesh = plsc.VectorSubcoreMesh(core_axis_name="c", subcore_axis_name="s")
    f = functools.partial(
        pl.kernel,
        mesh=mesh,
        compiler_params=pltpu.CompilerParams(needs_layout_passes=False),
        out_type=jax.ShapeDtypeStruct((BATCH, OUT_DIM), jnp.float32),
        scratch_types=[
            pltpu.VMEM((OUT_DIM,), jnp.int32),
            pltpu.VMEM((SLC,), jnp.int32),
            pltpu.VMEM((OUT_DIM,), jnp.uint32),
            pltpu.VMEM((OUT_DIM,), jnp.uint32),
            pltpu.VMEM((16, CW), jnp.float32),
            pltpu.VMEM((R * IN_DIM,), jnp.float32),
            pltpu.VMEM((R * IN_DIM,), jnp.float32),
            pltpu.VMEM((2, R, S), jnp.float32),
            pltpu.VMEM_SHARED((OUT_DIM,), jnp.int32),
            pltpu.VMEM_SHARED((OUT_DIM,), jnp.uint32),
            pltpu.VMEM_SHARED((OUT_DIM,), jnp.uint32),
            pltpu.SemaphoreType.DMA,
            pltpu.SemaphoreType.DMA,
        ],
    )(_tec_body)
    return f(x.reshape(-1), wt, idx_a, idx_b)


def kernel(x, weights, idx_a, idx_b):
    wt = jnp.transpose(weights)
    return _logic_layer_sc(x, wt, idx_a, idx_b)


# speedup vs baseline: 1.0899x; 1.0899x over previous
import functools

import jax
import jax.numpy as jnp
from jax import lax
from jax.experimental import pallas as pl
from jax.experimental.pallas import tpu as pltpu
from jax.experimental.pallas import tpu_sc as plsc

IN_DIM = 4096
OUT_DIM = 8192
BATCH = 4096

NC = 2
NS = 16
NW = NC * NS
L = 16

ROWS_PER_TILE = BATCH // NW
R = 8
NBLK = ROWS_PER_TILE // R
S = 2048
NSTRIP = OUT_DIM // S
CPS = S // L
CW = 128
NCW = OUT_DIM // CW


def _round_bf16_bits(v):
    bits = plsc.bitcast(v, jnp.uint32)
    return lax.shift_right_logical(bits + jnp.uint32(0x8000),
                                   jnp.uint32(16))


def _coef_body(w_cols):
    m = w_cols[0]
    for g in range(1, 16):
        m = jnp.maximum(m, w_cols[g])
    e = [jnp.exp(w_cols[g] - m) for g in range(16)]
    s = e[0]
    for g in range(1, 16):
        s = s + e[g]
    inv = 1.0 / s
    ca = (e[2] + e[3]) + (e[6] + e[7]) - (e[8] + e[9]) - (e[12] + e[13])
    cb = (e[4] + e[5]) + (e[6] + e[7]) - (e[8] + e[9]) - (e[10] + e[11])
    cab = (e[1] - e[2]) - (e[4] + e[7]) - 2.0 * (e[6] - e[9]) \
        + (e[8] + e[11]) + (e[13] - e[14])
    c0 = (e[8] + e[9]) + (e[10] + e[11]) + (e[12] + e[13]) + (e[14] + e[15])
    return c0 * inv, ca * inv, cb * inv, cab * inv


def _unpack_pair(pk):
    lo = plsc.bitcast(lax.shift_left(pk, jnp.uint32(16)), jnp.float32)
    hi = plsc.bitcast(pk & jnp.uint32(0xFFFF0000), jnp.float32)
    return lo, hi


SLC = OUT_DIM // NS


def _tec_body(x_hbm, wt_hbm, ia_hbm, ib_hbm, out_hbm,
              iab_v, stage_v, cp0_v, cp1_v, wbuf_v, xblk2_v, obuf_v,
              iab_s, cp0_s, cp1_s, sem_x, sem_o):
    cid = lax.axis_index("c")
    sid = lax.axis_index("s")
    wid = sid * NC + cid
    row0 = wid * ROWS_PER_TILE

    pltpu.async_copy(x_hbm.at[pl.ds(row0, R)], xblk2_v.at[0], sem_x)

    off = sid * SLC

    pltpu.sync_copy(ia_hbm.at[pl.ds(off, SLC)], iab_v.at[pl.ds(off, SLC)])
    pltpu.sync_copy(ib_hbm.at[pl.ds(off, SLC)], stage_v)

    def pack_idx(j, carry):
        ds = pl.ds(off + j * L, L)
        iab_v[ds] = iab_v[ds] | \
            lax.shift_left(stage_v[pl.ds(j * L, L)], 16)
        return carry

    lax.fori_loop(0, SLC // L, pack_idx, 0)
    pltpu.sync_copy(iab_v.at[pl.ds(off, SLC)], iab_s.at[pl.ds(off, SLC)])

    def coef_chunk(k, carry):
        c = sid * (SLC // CW) + k
        pltpu.sync_copy(wt_hbm.at[:, pl.ds(c * CW, CW)], wbuf_v)

        def coef_group(j, carry2):
            w_cols = [wbuf_v[g, pl.ds(j * L, L)] for g in range(16)]
            c0, ca, cb, cab = _coef_body(w_cols)
            base = pl.ds(c * CW + j * L, L)
            cp0_v[base] = _round_bf16_bits(c0) | \
                lax.shift_left(_round_bf16_bits(ca), jnp.uint32(16))
            cp1_v[base] = _round_bf16_bits(cb) | \
                lax.shift_left(_round_bf16_bits(cab), jnp.uint32(16))
            return carry2

        lax.fori_loop(0, CW // L, coef_group, 0)
        return carry

    lax.fori_loop(0, SLC // CW, coef_chunk, 0)
    pltpu.sync_copy(cp0_v.at[pl.ds(off, SLC)], cp0_s.at[pl.ds(off, SLC)])
    pltpu.sync_copy(cp1_v.at[pl.ds(off, SLC)], cp1_s.at[pl.ds(off, SLC)])

    plsc.subcore_barrier()

    pltpu.sync_copy(iab_s, iab_v)
    pltpu.sync_copy(cp0_s, cp0_v)
    pltpu.sync_copy(cp1_s, cp1_v)


    def row_block(blk, carry):
        px = blk & 1
        rbase = row0 + blk * R
        pltpu.make_async_copy(
            x_hbm.at[pl.ds(rbase, R)], xblk2_v.at[px], sem_x).wait()

        @pl.when(blk + 1 < NBLK)
        def _():
            pltpu.async_copy(
                x_hbm.at[pl.ds(rbase + R, R)], xblk2_v.at[1 - px], sem_x)

        pxv = jnp.broadcast_to(px, (L,)).astype(jnp.int32)

        def strip(st, carry2):
            pb = st & 1
            gst = blk * NSTRIP + st
            obase = st * S

            @pl.when(gst >= 2)
            def _():
                pltpu.make_async_copy(
                    obuf_v.at[pb],
                    out_hbm.at[pl.ds(rbase, R), pl.ds(obase, S)],
                    sem_o).wait()

            @plsc.parallel_loop(0, CPS, 1, unroll=2)
            def chunk(oc):
                ds = pl.ds(obase + oc * L, L)
                pk = iab_v[ds]
                ia = pk & jnp.int32(0xFFFF)
                ib = lax.shift_right_logical(pk, 16)
                c0, ca = _unpack_pair(cp0_v[ds])
                cb, cab = _unpack_pair(cp1_v[ds])
                for r in range(R):
                    rv = jnp.full((L,), r, dtype=jnp.int32)
                    a = plsc.load_gather(xblk2_v, [pxv, rv, ia])
                    b = plsc.load_gather(xblk2_v, [pxv, rv, ib])
                    obuf_v[pb, r, pl.ds(oc * L, L)] = \
                        (c0 + ca * a) + (cb + cab * a) * b

            pltpu.async_copy(
                obuf_v.at[pb],
                out_hbm.at[pl.ds(rbase, R), pl.ds(obase, S)],
                sem_o)
            return carry2

        lax.fori_loop(0, NSTRIP, strip, 0)
        return carry

    lax.fori_loop(0, NBLK, row_block, 0)

    for _ in range(2):
        pltpu.make_async_copy(
            obuf_v.at[0],
            out_hbm.at[pl.ds(row0, R), pl.ds(0, S)],
            sem_o).wait()


@jax.jit
def _logic_layer_sc(x, wt, idx_a, idx_b):
    mesh = plsc.VectorSubcoreMesh(core_axis_name="c", subcore_axis_name="s")
    f = functools.partial(
        pl.kernel,
        mesh=mesh,
        compiler_params=pltpu.CompilerParams(needs_layout_passes=False),
        out_type=jax.ShapeDtypeStruct((BATCH, OUT_DIM), jnp.float32),
        scratch_types=[
            pltpu.VMEM((OUT_DIM,), jnp.int32),
            pltpu.VMEM((SLC,), jnp.int32),
            pltpu.VMEM((OUT_DIM,), jnp.uint32),
            pltpu.VMEM((OUT_DIM,), jnp.uint32),
            pltpu.VMEM((16, CW), jnp.float32),
            pltpu.VMEM((2, R, IN_DIM), jnp.float32),
            pltpu.VMEM((2, R, S), jnp.float32),
            pltpu.VMEM_SHARED((OUT_DIM,), jnp.int32),
            pltpu.VMEM_SHARED((OUT_DIM,), jnp.uint32),
            pltpu.VMEM_SHARED((OUT_DIM,), jnp.uint32),
            pltpu.SemaphoreType.DMA,
            pltpu.SemaphoreType.DMA,
        ],
    )(_tec_body)
    return f(x, wt, idx_a, idx_b)


def kernel(x, weights, idx_a, idx_b):
    wt = jnp.transpose(weights)
    return _logic_layer_sc(x, wt, idx_a, idx_b)
